# baseline (device time: 79700 ns/iter reference)
import jax
import jax.numpy as jnp
from jax import lax
from jax.experimental import pallas as pl
from jax.experimental.pallas import tpu as pltpu

N_DEV = 16


def kernel(x, w_mat):
    m_per, k = x.shape
    _, n = w_mat.shape
    n_per = n // N_DEV
    m_out = N_DEV * m_per

    my = lax.axis_index("i")
    perm = (my + jnp.arange(N_DEV, dtype=jnp.int32)) % N_DEV

    def body(perm_ref, x_ref, w_ref, out_ref, comm, send_sems, recv_sems):
        j = pl.program_id(0)
        me = lax.axis_index("i")

        @pl.when(j == 0)
        def _barrier():
            barrier = pltpu.get_barrier_semaphore()
            for p in range(1, N_DEV):
                pl.semaphore_signal(
                    barrier,
                    inc=1,
                    device_id=((me + p) % N_DEV,),
                    device_id_type=pl.DeviceIdType.MESH,
                )
            pl.semaphore_wait(barrier, N_DEV - 1)

        y = jnp.dot(x_ref[...], w_ref[...], preferred_element_type=jnp.float32)
        y = jnp.maximum(y, 0.0).astype(jnp.bfloat16)

        @pl.when(j == 0)
        def _local():
            out_ref[pl.ds(me * m_per, m_per), :] = y

        @pl.when(j > 0)
        def _send():
            slot = j - 1
            comm[slot] = y
            rdma = pltpu.make_async_remote_copy(
                src_ref=comm.at[slot],
                dst_ref=out_ref.at[pl.ds(me * m_per, m_per), :],
                send_sem=send_sems.at[slot],
                recv_sem=recv_sems.at[slot],
                device_id=(perm_ref[j],),
                device_id_type=pl.DeviceIdType.MESH,
            )
            rdma.start()

        @pl.when(j == N_DEV - 1)
        def _drain():
            for t in range(N_DEV - 1):
                desc = pltpu.make_async_remote_copy(
                    src_ref=comm.at[t],
                    dst_ref=out_ref.at[pl.ds(0, m_per), :],
                    send_sem=send_sems.at[t],
                    recv_sem=recv_sems.at[t],
                    device_id=((me + 1) % N_DEV,),
                    device_id_type=pl.DeviceIdType.MESH,
                )
                desc.wait_send()
                desc.wait_recv()

    grid_spec = pltpu.PrefetchScalarGridSpec(
        num_scalar_prefetch=1,
        grid=(N_DEV,),
        in_specs=[
            pl.BlockSpec((m_per, k), lambda j, perm_ref: (0, 0)),
            pl.BlockSpec((k, n_per), lambda j, perm_ref: (0, perm_ref[j])),
        ],
        out_specs=pl.BlockSpec((m_out, n_per), lambda j, perm_ref: (0, 0)),
        scratch_shapes=[
            pltpu.VMEM((N_DEV - 1, m_per, n_per), jnp.bfloat16),
            pltpu.SemaphoreType.DMA((N_DEV - 1,)),
            pltpu.SemaphoreType.DMA((N_DEV - 1,)),
        ],
    )

    return pl.pallas_call(
        body,
        grid_spec=grid_spec,
        out_shape=jax.ShapeDtypeStruct((m_out, n_per), jnp.bfloat16),
        compiler_params=pltpu.CompilerParams(
            dimension_semantics=("arbitrary",),
            collective_id=0,
        ),
    )(perm, x, w_mat)
